# baseline (speedup 1.0000x reference)
import functools

import numpy as np

import jax
import jax.numpy as jnp
from jax import lax
from jax.experimental import pallas as pl
from jax.experimental.pallas import tpu as pltpu
from jax.experimental.pallas import tpu_sc as plsc

B = 2
C = 128
N = 10000
K = 16
OUT_C = 128
GROUPS = 4
BN = B * N

NC = 2
NS = 16
NW = NC * NS
NPW = 640
CH = 8
ROWS = CH * K
FULL_CHUNKS = NPW // CH
LASTW_CHUNKS = (BN - (NW - 1) * NPW) // CH
L = 16
CW = C // 2


def _sc_body(xt_hbm, i0_hbm, i1_hbm, out_hbm,
             i0v, i1v, r0, r1, ov,
             gsem0, gsem1, osem0, osem1):
    gsems = (gsem0, gsem1)
    osems = (osem0, osem1)
    wid = lax.axis_index("s") * NC + lax.axis_index("c")
    obase = wid * NPW
    nchunk = jnp.where(wid == NW - 1, LASTW_CHUNKS, FULL_CHUNKS)
    r0b = r0.bitcast(jnp.bfloat16)
    r1b = r1.bitcast(jnp.bfloat16)
    ovb = ov.bitcast(jnp.bfloat16)

    pltpu.sync_copy(i0_hbm.at[pl.ds(wid * (NPW * K), NPW * K)], i0v)
    pltpu.sync_copy(i1_hbm.at[pl.ds(wid * (NPW * K), NPW * K)], i1v)

    def gather_descs(c, s):
        off = c * ROWS
        d0 = pltpu.make_async_copy(
            xt_hbm.at[i0v.at[pl.ds(off, ROWS)]], r0.at[s], gsems[s])
        d1 = pltpu.make_async_copy(
            xt_hbm.at[i1v.at[pl.ds(off, ROWS)]], r1.at[s], gsems[s])
        return d0, d1

    def gather_start(c, s):
        d0, d1 = gather_descs(c, s)
        d0.start()
        d1.start()

    def gather_wait(c, s):
        d0, d1 = gather_descs(c, s)
        d0.wait()
        d1.wait()

    def store_desc(c, s):
        return pltpu.make_async_copy(
            ov.at[s], out_hbm.at[pl.ds(obase + c * CH, CH)], osems[s])

    def compute(c, s):
        @pl.loop(0, CH)
        def _(n):
            rbase = n * (2 * K)
            for g in range(CW // L):
                sl = pl.ds(g * L, L)
                a = (r0b[s, pl.ds(pl.multiple_of(rbase, 2), 2), sl]
                     - r1b[s, pl.ds(pl.multiple_of(rbase, 2), 2), sl])
                for kk in range(1, K):
                    rr = pl.multiple_of(rbase + 2 * kk, 2)
                    a = jnp.maximum(
                        a, r0b[s, pl.ds(rr, 2), sl] - r1b[s, pl.ds(rr, 2), sl])
                orow = pl.multiple_of(2 * n, 2)
                ovb[s, pl.ds(orow, 2), sl] = a

    gather_start(0, 0)
    gather_start(1, 1)

    @pl.loop(0, nchunk, step=2)
    def _(c0):
        for s in range(2):
            c = c0 + s
            gather_wait(c, s)

            @pl.when(c >= 2)
            def _():
                store_desc(c - 2, s).wait()

            compute(c, s)
            store_desc(c, s).start()

            @pl.when(c + 2 < nchunk)
            def _():
                gather_start(c + 2, s)

    store_desc(nchunk - 2, 0).wait()
    store_desc(nchunk - 1, 1).wait()


def _sc_maxrel(xt, i0, i1):
    mesh = plsc.VectorSubcoreMesh(core_axis_name="c", subcore_axis_name="s")
    kfn = functools.partial(
        pl.kernel,
        mesh=mesh,
        out_type=jax.ShapeDtypeStruct((NW * NPW, CW), jnp.int32),
        scratch_types=[
            pltpu.VMEM((NPW * K,), jnp.int32),
            pltpu.VMEM((NPW * K,), jnp.int32),
            pltpu.VMEM((2, ROWS, C), jnp.int32),
            pltpu.VMEM((2, ROWS, C), jnp.int32),
            pltpu.VMEM((2, CH, CW), jnp.int32),
            pltpu.SemaphoreType.DMA,
            pltpu.SemaphoreType.DMA,
            pltpu.SemaphoreType.DMA,
            pltpu.SemaphoreType.DMA,
        ],
    )(_sc_body)
    return kfn(xt, i0, i1)


def _conv_body(x_ref, xj_ref, ax_ref, aj_ref, b_ref, o_ref):
    xb = x_ref[0]
    xjb = xj_ref[0]
    acc = lax.dot_general(ax_ref[...], xb, (((1,), (0,)), ((), ())),
                          preferred_element_type=jnp.float32)
    acc = acc + lax.dot_general(aj_ref[...].astype(jnp.bfloat16), xjb,
                                (((1,), (1,)), ((), ())),
                                preferred_element_type=jnp.float32)
    o_ref[0] = jnp.maximum(acc + b_ref[...], 0.0)


def _conv(xcn, xj_nc, ax, aj, b2):
    nt = 2048
    grid = (B, pl.cdiv(N, nt))
    return pl.pallas_call(
        _conv_body,
        grid=grid,
        in_specs=[
            pl.BlockSpec((1, C, nt), lambda bb, t: (bb, 0, t)),
            pl.BlockSpec((1, nt, C), lambda bb, t: (bb, t, 0)),
            pl.BlockSpec((OUT_C, C), lambda bb, t: (0, 0)),
            pl.BlockSpec((OUT_C, C), lambda bb, t: (0, 0)),
            pl.BlockSpec((OUT_C, 1), lambda bb, t: (0, 0)),
        ],
        out_specs=pl.BlockSpec((1, OUT_C, nt), lambda bb, t: (bb, 0, t)),
        out_shape=jax.ShapeDtypeStruct((B, OUT_C, N), jnp.float32),
    )(xcn, xj_nc, ax, aj, b2)


def kernel(x, edge_index, W, b):
    xsq = x[:, :, :, 0]
    xb16 = jnp.transpose(xsq.astype(jnp.bfloat16), (0, 2, 1))
    xp = lax.bitcast_convert_type(
        xb16.reshape(BN, CW, 2), jnp.int32)
    xt = jnp.concatenate([xp, xp], axis=1)
    offs = (jnp.arange(B, dtype=jnp.int32) * N).reshape(1, B, 1, 1)
    ef = edge_index + offs
    pad = jnp.zeros((NW * NPW - BN) * K, jnp.int32)
    i0 = jnp.concatenate([ef[0].reshape(BN * K), pad])
    i1 = jnp.concatenate([ef[1].reshape(BN * K), pad])

    xj_w = _sc_maxrel(xt, i0, i1)
    xj = lax.bitcast_convert_type(
        xj_w[:BN].reshape(B, N, CW), jnp.bfloat16).reshape(B, N, C)

    wr = W.reshape(GROUPS, OUT_C // GROUPS, C // GROUPS, 2)
    ax = jax.scipy.linalg.block_diag(*[wr[g, :, :, 0] for g in range(GROUPS)])
    aj = jax.scipy.linalg.block_diag(*[wr[g, :, :, 1] for g in range(GROUPS)])

    out = _conv(xsq, xj, ax, aj, b.reshape(OUT_C, 1))
    return out[..., None]


# speedup vs baseline: 1.3896x; 1.3896x over previous
import functools

import jax
import jax.numpy as jnp
from jax import lax
from jax.experimental import pallas as pl
from jax.experimental.pallas import tpu as pltpu
from jax.experimental.pallas import tpu_sc as plsc

B = 2
C = 128
N = 10000
K = 16
OUT_C = 128
GROUPS = 4
BN = B * N

NC = 2
NS = 16
NW = NC * NS
NPW = N // NS
CH = 5
ROWS = CH * K
NCHUNK = NPW // CH
L = 16
STG = 632
STG_LAST = N - (NS - 1) * STG


def _sc_body(xt_hbm, i0_hbm, i1_hbm, out_hbm,
             table, i0v0, i0v1, i1v0, i1v1, r0, r1, ov,
             gsem0, gsem1, osem0, osem1, isem0, isem1):
    i0vs = (i0v0, i0v1)
    i1vs = (i1v0, i1v1)
    gsems = (gsem0, gsem1)
    osems = (osem0, osem1)
    isems = (isem0, isem1)
    cc = lax.axis_index("c")
    ss = lax.axis_index("s")
    obase = (cc * N + ss * NPW) * C

    @pl.when(ss < NS - 1)
    def _():
        pltpu.sync_copy(xt_hbm.at[pl.ds(cc * N + ss * STG, STG)],
                        table.at[pl.ds(ss * STG, STG)])

    @pl.when(ss == NS - 1)
    def _():
        pltpu.sync_copy(
            xt_hbm.at[pl.ds(cc * N + (NS - 1) * STG, STG_LAST)],
            table.at[pl.ds((NS - 1) * STG, STG_LAST)])

    ibase = (cc * N + ss * NPW) * K

    def idx_descs(c, s):
        off = ibase + c * ROWS
        d0 = pltpu.make_async_copy(
            i0_hbm.at[pl.ds(off, ROWS)], i0vs[s], isems[s])
        d1 = pltpu.make_async_copy(
            i1_hbm.at[pl.ds(off, ROWS)], i1vs[s], isems[s])
        return d0, d1

    def idx_start(c, s):
        d0, d1 = idx_descs(c, s)
        d0.start()
        d1.start()

    def idx_wait(c, s):
        d0, d1 = idx_descs(c, s)
        d0.wait()
        d1.wait()

    plsc.subcore_barrier()

    def gather_descs(c, s):
        d0 = pltpu.make_async_copy(
            table.at[i0vs[s]], r0.at[s], gsems[s])
        d1 = pltpu.make_async_copy(
            table.at[i1vs[s]], r1.at[s], gsems[s])
        return d0, d1

    def gather_start(c, s):
        d0, d1 = gather_descs(c, s)
        d0.start()
        d1.start()

    def gather_wait(c, s):
        d0, d1 = gather_descs(c, s)
        d0.wait()
        d1.wait()

    def store_desc(c, s):
        return pltpu.make_async_copy(
            ov.at[s], out_hbm.at[pl.ds(obase + c * (CH * C), CH * C)],
            osems[s])

    def compute(c, s):
        @pl.loop(0, CH)
        def _(n):
            row = n * K
            for g in range(C // L):
                sl = pl.ds(g * L, L)
                a = r0[s, row, sl] - r1[s, row, sl]
                for kk in range(1, K):
                    a = jnp.maximum(a, r0[s, row + kk, sl] - r1[s, row + kk, sl])
                ov[s, pl.ds(n * C + g * L, L)] = a

    idx_start(0, 0)
    idx_start(1, 1)
    idx_wait(0, 0)
    gather_start(0, 0)
    idx_wait(1, 1)
    gather_start(1, 1)

    @pl.loop(0, NCHUNK - 1, step=2)
    def _(c0):
        for s in range(2):
            c = c0 + s
            gather_wait(c, s)

            @pl.when(c + 2 < NCHUNK)
            def _():
                idx_start(c + 2, s)

            @pl.when(c >= 2)
            def _():
                store_desc(c - 2, s).wait()

            compute(c, s)
            store_desc(c, s).start()

            @pl.when(c + 2 < NCHUNK)
            def _():
                idx_wait(c + 2, s)
                gather_start(c + 2, s)

    last = NCHUNK - 1
    gather_wait(last, 0)
    store_desc(last - 2, 0).wait()
    compute(last, 0)
    store_desc(last, 0).start()
    store_desc(last - 1, 1).wait()
    store_desc(last, 0).wait()


def _sc_maxrel(xt, i0, i1):
    mesh = plsc.VectorSubcoreMesh(core_axis_name="c", subcore_axis_name="s")
    kfn = functools.partial(
        pl.kernel,
        mesh=mesh,
        out_type=jax.ShapeDtypeStruct((BN * C,), jnp.float32),
        scratch_types=[
            pltpu.VMEM_SHARED((N, C), jnp.float32),
            pltpu.VMEM((ROWS,), jnp.int32),
            pltpu.VMEM((ROWS,), jnp.int32),
            pltpu.VMEM((ROWS,), jnp.int32),
            pltpu.VMEM((ROWS,), jnp.int32),
            pltpu.VMEM((2, ROWS, C), jnp.float32),
            pltpu.VMEM((2, ROWS, C), jnp.float32),
            pltpu.VMEM((2, CH * C), jnp.float32),
            pltpu.SemaphoreType.DMA,
            pltpu.SemaphoreType.DMA,
            pltpu.SemaphoreType.DMA,
            pltpu.SemaphoreType.DMA,
            pltpu.SemaphoreType.DMA,
            pltpu.SemaphoreType.DMA,
        ],
    )(_sc_body)
    return kfn(xt, i0, i1)


def _conv_body(x_ref, xj_ref, ax_ref, aj_ref, b_ref, o_ref):
    xb = x_ref[0]
    xjb = xj_ref[0]
    acc = lax.dot_general(ax_ref[...], xb, (((1,), (0,)), ((), ())),
                          preferred_element_type=jnp.float32)
    acc = acc + lax.dot_general(aj_ref[...], xjb, (((1,), (1,)), ((), ())),
                                preferred_element_type=jnp.float32)
    o_ref[0] = jnp.maximum(acc + b_ref[...], 0.0)


def _conv(xcn, xj_nc, ax, aj, b2):
    nt = 2048
    grid = (B, pl.cdiv(N, nt))
    return pl.pallas_call(
        _conv_body,
        grid=grid,
        in_specs=[
            pl.BlockSpec((1, C, nt), lambda bb, t: (bb, 0, t)),
            pl.BlockSpec((1, nt, C), lambda bb, t: (bb, t, 0)),
            pl.BlockSpec((OUT_C, C), lambda bb, t: (0, 0)),
            pl.BlockSpec((OUT_C, C), lambda bb, t: (0, 0)),
            pl.BlockSpec((OUT_C, 1), lambda bb, t: (0, 0)),
        ],
        out_specs=pl.BlockSpec((1, OUT_C, nt), lambda bb, t: (bb, 0, t)),
        out_shape=jax.ShapeDtypeStruct((B, OUT_C, N), jnp.float32),
    )(xcn, xj_nc, ax, aj, b2)


def kernel(x, edge_index, W, b):
    xsq = x[:, :, :, 0]
    xt = jnp.transpose(xsq, (0, 2, 1)).reshape(BN, C)
    i0 = edge_index[0].reshape(BN * K)
    i1 = edge_index[1].reshape(BN * K)

    xj = _sc_maxrel(xt, i0, i1).reshape(B, N, C)

    wr = W.reshape(GROUPS, OUT_C // GROUPS, C // GROUPS, 2)
    ax = jax.scipy.linalg.block_diag(*[wr[g, :, :, 0] for g in range(GROUPS)])
    aj = jax.scipy.linalg.block_diag(*[wr[g, :, :, 1] for g in range(GROUPS)])

    out = _conv(xsq, xj, ax, aj, b.reshape(OUT_C, 1))
    return out[..., None]


# speedup vs baseline: 1.6053x; 1.1552x over previous
import functools

import jax
import jax.numpy as jnp
from jax import lax
from jax.experimental import pallas as pl
from jax.experimental.pallas import tpu as pltpu
from jax.experimental.pallas import tpu_sc as plsc

B = 2
C = 128
N = 10000
K = 16
OUT_C = 128
GROUPS = 4
BN = B * N

NC = 2
NS = 16
NW = NC * NS
NPW = 640
CH = 8
ROWS = CH * K
FULL_CHUNKS = NPW // CH
LASTW_CHUNKS = (BN - (NW - 1) * NPW) // CH
L = 16


def _sc_body(xt_hbm, i0_hbm, i1_hbm, out_hbm,
             i0v, i1v, r0, r1, ov,
             gsem0, gsem1, osem0, osem1):
    gsems = (gsem0, gsem1)
    osems = (osem0, osem1)
    wid = lax.axis_index("s") * NC + lax.axis_index("c")
    obase = wid * (NPW * C)
    nchunk = jnp.where(wid == NW - 1, LASTW_CHUNKS, FULL_CHUNKS)

    pltpu.sync_copy(i0_hbm.at[pl.ds(wid * (NPW * K), NPW * K)], i0v)
    pltpu.sync_copy(i1_hbm.at[pl.ds(wid * (NPW * K), NPW * K)], i1v)

    def gather_descs(c, s):
        off = c * ROWS
        d0 = pltpu.make_async_copy(
            xt_hbm.at[i0v.at[pl.ds(off, ROWS)]], r0.at[s], gsems[s])
        d1 = pltpu.make_async_copy(
            xt_hbm.at[i1v.at[pl.ds(off, ROWS)]], r1.at[s], gsems[s])
        return d0, d1

    def gather_start(c, s):
        d0, d1 = gather_descs(c, s)
        d0.start()
        d1.start()

    def gather_wait(c, s):
        d0, d1 = gather_descs(c, s)
        d0.wait()
        d1.wait()

    def store_desc(c, s):
        return pltpu.make_async_copy(
            ov.at[s], out_hbm.at[pl.ds(obase + c * (CH * C), CH * C)],
            osems[s])

    def compute(c, s):
        @pl.loop(0, CH)
        def _(n):
            row = n * K
            for g in range(C // L):
                sl = pl.ds(g * L, L)
                a = r0[s, row, sl] - r1[s, row, sl]
                for kk in range(1, K):
                    a = jnp.maximum(a, r0[s, row + kk, sl] - r1[s, row + kk, sl])
                ov[s, pl.ds(n * C + g * L, L)] = a

    gather_start(0, 0)
    gather_start(1, 1)

    @pl.loop(0, nchunk, step=2)
    def _(c0):
        for s in range(2):
            c = c0 + s
            gather_wait(c, s)

            @pl.when(c >= 2)
            def _():
                store_desc(c - 2, s).wait()

            compute(c, s)
            store_desc(c, s).start()

            @pl.when(c + 2 < nchunk)
            def _():
                gather_start(c + 2, s)

    store_desc(nchunk - 2, 0).wait()
    store_desc(nchunk - 1, 1).wait()


def _sc_maxrel(xt, i0, i1):
    mesh = plsc.VectorSubcoreMesh(core_axis_name="c", subcore_axis_name="s")
    kfn = functools.partial(
        pl.kernel,
        mesh=mesh,
        out_type=jax.ShapeDtypeStruct((NW * NPW * C,), jnp.float32),
        scratch_types=[
            pltpu.VMEM((NPW * K,), jnp.int32),
            pltpu.VMEM((NPW * K,), jnp.int32),
            pltpu.VMEM((2, ROWS, C), jnp.float32),
            pltpu.VMEM((2, ROWS, C), jnp.float32),
            pltpu.VMEM((2, CH * C), jnp.float32),
            pltpu.SemaphoreType.DMA,
            pltpu.SemaphoreType.DMA,
            pltpu.SemaphoreType.DMA,
            pltpu.SemaphoreType.DMA,
        ],
    )(_sc_body)
    return kfn(xt, i0, i1)


def _conv_body(x_ref, xj_ref, ax_ref, aj_ref, b_ref, o_ref):
    xb = x_ref[0]
    xjb = xj_ref[0]
    acc = lax.dot_general(ax_ref[...], xb, (((1,), (0,)), ((), ())),
                          preferred_element_type=jnp.float32)
    acc = acc + lax.dot_general(aj_ref[...], xjb, (((1,), (1,)), ((), ())),
                                preferred_element_type=jnp.float32)
    o_ref[0] = jnp.maximum(acc + b_ref[...], 0.0)


def _conv(xcn, xj_nc, ax, aj, b2):
    nt = 2048
    grid = (B, pl.cdiv(N, nt))
    return pl.pallas_call(
        _conv_body,
        grid=grid,
        in_specs=[
            pl.BlockSpec((1, C, nt), lambda bb, t: (bb, 0, t)),
            pl.BlockSpec((1, nt, C), lambda bb, t: (bb, t, 0)),
            pl.BlockSpec((OUT_C, C), lambda bb, t: (0, 0)),
            pl.BlockSpec((OUT_C, C), lambda bb, t: (0, 0)),
            pl.BlockSpec((OUT_C, 1), lambda bb, t: (0, 0)),
        ],
        out_specs=pl.BlockSpec((1, OUT_C, nt), lambda bb, t: (bb, 0, t)),
        out_shape=jax.ShapeDtypeStruct((B, OUT_C, N), jnp.float32),
    )(xcn, xj_nc, ax, aj, b2)


def kernel(x, edge_index, W, b):
    xsq = x[:, :, :, 0]
    xt = jnp.transpose(xsq, (0, 2, 1)).reshape(BN, C)
    offs = (jnp.arange(B, dtype=jnp.int32) * N).reshape(1, B, 1, 1)
    ef = edge_index + offs
    pad = jnp.zeros((NW * NPW - BN) * K, jnp.int32)
    i0 = jnp.concatenate([ef[0].reshape(BN * K), pad])
    i1 = jnp.concatenate([ef[1].reshape(BN * K), pad])

    xj = _sc_maxrel(xt, i0, i1)
    xj = xj[:BN * C].reshape(B, N, C)

    wr = W.reshape(GROUPS, OUT_C // GROUPS, C // GROUPS, 2)
    ax = jax.scipy.linalg.block_diag(*[wr[g, :, :, 0] for g in range(GROUPS)])
    aj = jax.scipy.linalg.block_diag(*[wr[g, :, :, 1] for g in range(GROUPS)])

    out = _conv(xsq, xj, ax, aj, b.reshape(OUT_C, 1))
    return out[..., None]


# speedup vs baseline: 1.6773x; 1.0448x over previous
import functools

import jax
import jax.numpy as jnp
from jax import lax
from jax.experimental import pallas as pl
from jax.experimental.pallas import tpu as pltpu
from jax.experimental.pallas import tpu_sc as plsc

B = 2
C = 128
N = 10000
K = 16
OUT_C = 128
GROUPS = 4
BN = B * N

NC = 2
NS = 16
NW = NC * NS
CH = 8
ROWS = CH * K
BCHUNKS = BN // CH
BASE_CHUNKS = BCHUNKS // NW
XTRA = BCHUNKS - BASE_CHUNKS * NW
MAXN = (BASE_CHUNKS + 1) * CH
L = 16


def _sc_body(xt_hbm, i0_hbm, i1_hbm, out_hbm,
             i0v, i1v, r0, r1, ov,
             gsem0, gsem1, osem0, osem1):
    gsems = (gsem0, gsem1)
    osems = (osem0, osem1)
    wid = lax.axis_index("s") * NC + lax.axis_index("c")
    start = wid * (BASE_CHUNKS * CH) + CH * jnp.minimum(wid, XTRA)
    nchunk = jnp.where(wid < XTRA, BASE_CHUNKS + 1, BASE_CHUNKS)
    obase = start * C
    ibase = start * K

    @pl.when(wid < XTRA)
    def _():
        pltpu.sync_copy(i0_hbm.at[pl.ds(ibase, MAXN * K)], i0v)
        pltpu.sync_copy(i1_hbm.at[pl.ds(ibase, MAXN * K)], i1v)

    @pl.when(wid >= XTRA)
    def _():
        nk = BASE_CHUNKS * CH * K
        pltpu.sync_copy(i0_hbm.at[pl.ds(ibase, nk)], i0v.at[pl.ds(0, nk)])
        pltpu.sync_copy(i1_hbm.at[pl.ds(ibase, nk)], i1v.at[pl.ds(0, nk)])

    def gather_descs(c, s):
        off = c * ROWS
        d0 = pltpu.make_async_copy(
            xt_hbm.at[i0v.at[pl.ds(off, ROWS)]], r0.at[s], gsems[s])
        d1 = pltpu.make_async_copy(
            xt_hbm.at[i1v.at[pl.ds(off, ROWS)]], r1.at[s], gsems[s])
        return d0, d1

    def gather_start(c, s):
        d0, d1 = gather_descs(c, s)
        d0.start()
        d1.start()

    def gather_wait(c, s):
        d0, d1 = gather_descs(c, s)
        d0.wait()
        d1.wait()

    def store_desc(c, s):
        return pltpu.make_async_copy(
            ov.at[s], out_hbm.at[pl.ds(obase + c * (CH * C), CH * C)],
            osems[s])

    def compute(c, s):
        @pl.loop(0, CH)
        def _(n):
            row = n * K
            for g in range(C // L):
                sl = pl.ds(g * L, L)
                a = r0[s, row, sl] - r1[s, row, sl]
                for kk in range(1, K):
                    a = jnp.maximum(a, r0[s, row + kk, sl] - r1[s, row + kk, sl])
                ov[s, pl.ds(n * C + g * L, L)] = a

    gather_start(0, 0)
    gather_start(1, 1)

    odd = lax.rem(nchunk, 2)
    twon = nchunk - odd

    @pl.loop(0, twon, step=2)
    def _(c0):
        for s in range(2):
            c = c0 + s
            gather_wait(c, s)

            @pl.when(c >= 2)
            def _():
                store_desc(c - 2, s).wait()

            compute(c, s)
            store_desc(c, s).start()

            @pl.when(c + 2 < nchunk)
            def _():
                gather_start(c + 2, s)

    @pl.when(odd == 1)
    def _():
        gather_wait(twon, 0)
        store_desc(twon - 2, 0).wait()
        compute(twon, 0)
        store_desc(twon, 0).start()
        store_desc(twon - 1, 1).wait()
        store_desc(twon, 0).wait()

    @pl.when(odd == 0)
    def _():
        store_desc(twon - 2, 0).wait()
        store_desc(twon - 1, 1).wait()


def _sc_maxrel(xt, i0, i1):
    mesh = plsc.VectorSubcoreMesh(core_axis_name="c", subcore_axis_name="s")
    kfn = functools.partial(
        pl.kernel,
        mesh=mesh,
        out_type=jax.ShapeDtypeStruct((BN * C,), jnp.float32),
        scratch_types=[
            pltpu.VMEM((MAXN * K,), jnp.int32),
            pltpu.VMEM((MAXN * K,), jnp.int32),
            pltpu.VMEM((2, ROWS, C), jnp.float32),
            pltpu.VMEM((2, ROWS, C), jnp.float32),
            pltpu.VMEM((2, CH * C), jnp.float32),
            pltpu.SemaphoreType.DMA,
            pltpu.SemaphoreType.DMA,
            pltpu.SemaphoreType.DMA,
            pltpu.SemaphoreType.DMA,
        ],
    )(_sc_body)
    return kfn(xt, i0, i1)


def _conv_body(x_ref, xj_ref, ax_ref, aj_ref, b_ref, o_ref):
    xb = x_ref[0]
    xjb = xj_ref[0]
    acc = lax.dot_general(ax_ref[...], xb, (((1,), (0,)), ((), ())),
                          preferred_element_type=jnp.float32)
    acc = acc + lax.dot_general(aj_ref[...], xjb, (((1,), (1,)), ((), ())),
                                preferred_element_type=jnp.float32)
    o_ref[0] = jnp.maximum(acc + b_ref[...], 0.0)


def _conv(xcn, xj_nc, ax, aj, b2):
    nt = 2048
    grid = (B, pl.cdiv(N, nt))
    return pl.pallas_call(
        _conv_body,
        grid=grid,
        in_specs=[
            pl.BlockSpec((1, C, nt), lambda bb, t: (bb, 0, t)),
            pl.BlockSpec((1, nt, C), lambda bb, t: (bb, t, 0)),
            pl.BlockSpec((OUT_C, C), lambda bb, t: (0, 0)),
            pl.BlockSpec((OUT_C, C), lambda bb, t: (0, 0)),
            pl.BlockSpec((OUT_C, 1), lambda bb, t: (0, 0)),
        ],
        out_specs=pl.BlockSpec((1, OUT_C, nt), lambda bb, t: (bb, 0, t)),
        out_shape=jax.ShapeDtypeStruct((B, OUT_C, N), jnp.float32),
    )(xcn, xj_nc, ax, aj, b2)


def kernel(x, edge_index, W, b):
    xsq = x[:, :, :, 0]
    xt = jnp.transpose(xsq, (0, 2, 1)).reshape(BN, C)
    offs = (jnp.arange(B, dtype=jnp.int32) * N).reshape(1, B, 1, 1)
    ef = edge_index + offs
    i0 = ef[0].reshape(BN * K)
    i1 = ef[1].reshape(BN * K)

    xj = _sc_maxrel(xt, i0, i1).reshape(B, N, C)

    wr = W.reshape(GROUPS, OUT_C // GROUPS, C // GROUPS, 2)
    ax = jax.scipy.linalg.block_diag(*[wr[g, :, :, 0] for g in range(GROUPS)])
    aj = jax.scipy.linalg.block_diag(*[wr[g, :, :, 1] for g in range(GROUPS)])

    out = _conv(xsq, xj, ax, aj, b.reshape(OUT_C, 1))
    return out[..., None]
